# Initial kernel scaffold; baseline (speedup 1.0000x reference)
import jax
import jax.numpy as jnp
from jax.experimental import pallas as pl


def kernel(x, edge_index, params):
    raise NotImplementedError("write your pallas kernel here")


# speedup vs baseline: 149.3636x; 149.3636x over previous
import functools

import jax
import jax.numpy as jnp
from jax import lax
from jax.experimental import pallas as pl
from jax.experimental.pallas import tpu as pltpu
from jax.experimental.pallas import tpu_sc as plsc
from jax._src.pallas import core as pallas_core
from jax._src.pallas.mosaic import core as tpu_core

N = 100000
NT = 102400
E = 6400000
NC, NS, L = 2, 16, 16
NW = NC * NS
NBLK = 49
BLKE = 4096
CH = 128
EPT = NBLK * BLKE
E_PAD = NW * EPT
ROWS_PT = NT // NS
B = 4096


def _prep_call(x, w5, as5, ad5):
  din = x.shape[1]

  def body(x_ref, w_ref, as_ref, ad_ref, t_ref, adt_ref):
    h = jnp.dot(x_ref[...], w_ref[...], preferred_element_type=jnp.float32)
    als = jnp.sum(h * as_ref[...], axis=1, keepdims=True)
    ald = jnp.sum(h * ad_ref[...], axis=1, keepdims=True)
    ones = jnp.ones((B, 1), jnp.float32)
    t_ref[...] = jnp.concatenate(
        [h, ones, als, ald, jnp.zeros((B, 8), jnp.float32)], axis=1)
    adt_ref[...] = ald

  return pl.pallas_call(
      body,
      grid=(NT // B,),
      in_specs=[
          pl.BlockSpec((B, din), lambda i: (i, 0)),
          pl.BlockSpec((din, 5), lambda i: (0, 0)),
          pl.BlockSpec((1, 5), lambda i: (0, 0)),
          pl.BlockSpec((1, 5), lambda i: (0, 0)),
      ],
      out_specs=[
          pl.BlockSpec((B, 16), lambda i: (i, 0)),
          pl.BlockSpec((B, 1), lambda i: (i, 0)),
      ],
      out_shape=[
          jax.ShapeDtypeStruct((NT, 16), jnp.float32),
          jax.ShapeDtypeStruct((NT, 1), jnp.float32),
      ],
  )(x, w5, as5, ad5)


def _finalize_call(acc, t, b5, do_relu):
  def body(acc_ref, t_ref, b_ref, o_ref):
    a0 = acc_ref[0]
    a1 = acc_ref[1]
    num = a0[:, :5] + a1[:, :5]
    den = a0[:, 5:6] + a1[:, 5:6]
    h = t_ref[:, :5]
    es = t_ref[:, 6:7] + t_ref[:, 7:8]
    ws = jnp.exp(jnp.where(es > 0, es, es * 0.2))
    num = num + ws * h
    den = den + ws
    o = num / den + b_ref[...]
    if do_relu:
      o = jnp.maximum(o, 0.0)
    i = pl.program_id(0)
    rid = i * B + lax.broadcasted_iota(jnp.int32, (B, 5), 0)
    o_ref[...] = jnp.where(rid < N, o, 0.0)

  return pl.pallas_call(
      body,
      grid=(NT // B,),
      in_specs=[
          pl.BlockSpec((2, B, 8), lambda i: (0, i, 0)),
          pl.BlockSpec((B, 16), lambda i: (i, 0)),
          pl.BlockSpec((1, 5), lambda i: (0, 0)),
      ],
      out_specs=pl.BlockSpec((B, 5), lambda i: (i, 0)),
      out_shape=jax.ShapeDtypeStruct((NT, 5), jnp.float32),
  )(acc, t, b5)


def _make_edge_kernel():
  mesh = plsc.VectorSubcoreMesh(core_axis_name="c", subcore_axis_name="s")
  shared = pallas_core.CoreMemorySpace(tpu_core.MemorySpace.VMEM_SHARED, mesh)

  @functools.partial(
      pl.kernel,
      out_type=jax.ShapeDtypeStruct((NC, NT, 8), jnp.float32),
      mesh=mesh,
      compiler_params=pltpu.CompilerParams(needs_layout_passes=False,
                                           use_tc_tiling_on_sc=False),
      scratch_types=[
          pltpu.VMEM((BLKE // CH, CH), jnp.int32),
          pltpu.VMEM((BLKE // CH, CH), jnp.int32),
          pltpu.VMEM((CH, 16), jnp.float32),
          pltpu.VMEM((CH, 16), jnp.float32),
          pltpu.VMEM((CH, 8), jnp.float32),
          pltpu.VMEM((CH, 8), jnp.float32),
          pltpu.VMEM((CH,), jnp.float32),
          pltpu.VMEM((CH,), jnp.float32),
          pltpu.VMEM((L,), jnp.float32),
          shared((NT, 8), jnp.float32),
          shared((NT,), jnp.float32),
          pltpu.SemaphoreType.DMA,
          pltpu.SemaphoreType.DMA,
          pltpu.SemaphoreType.DMA,
          pltpu.SemaphoreType.DMA,
          pltpu.SemaphoreType.DMA,
          pltpu.SemaphoreType.DMA,
      ],
  )
  def edge_kernel(t_hbm, adt_hbm, src_hbm, dst_hbm, acc_out,
                  srcb, dstb, rows0, rows1, out0, out1, adch0, adch1, wtmp,
                  acc_sh, adt_sh,
                  sem_g0, sem_g1, sem_a0, sem_a1, sem_s0, sem_s1):
    cid = lax.axis_index("c")
    sid = lax.axis_index("s")
    wid = cid * NS + sid
    iota = lax.iota(jnp.int32, L)
    shr3 = lax.shift_right_logical(iota, 3)
    col07 = lax.bitwise_and(iota, 7)
    zeros16 = jnp.zeros((L,), jnp.float32)
    col6 = jnp.full((L,), 6, jnp.int32)

    @pl.when(sid == 0)
    def _():
      pltpu.sync_copy(adt_hbm, adt_sh)

    for p in range(64):
      plsc.store_scatter(out0, [shr3 + 2 * p, col07], zeros16)

    def zero_body(j, carry):
      pltpu.sync_copy(out0, acc_sh.at[pl.ds(sid * ROWS_PT + j * CH, CH)])
      return carry

    lax.fori_loop(0, ROWS_PT // CH, zero_body, 0)
    plsc.subcore_barrier()

    row_base = wid * (EPT // CH)

    def compute_chunk(rows_r, adch_r, out_r):
      for g in range(BLKE // CH // 4):
        colg = g * L + iota
        asrc = plsc.load_gather(rows_r, [colg, col6])
        adst = adch_r[pl.ds(g * L, L)]
        e = asrc + adst
        e = jnp.where(e > 0, e, e * 0.2)
        wtmp[...] = jnp.exp(e)
        for p in range(8):
          rowv = g * L + 2 * p + shr3
          vals = plsc.load_gather(rows_r, [rowv, col07])
          wv = plsc.load_gather(wtmp, [2 * p + shr3])
          plsc.store_scatter(out_r, [rowv, col07], vals * wv)

    def block_body(b, carry):
      blk_row = row_base + b * (BLKE // CH)
      pltpu.sync_copy(src_hbm.at[pl.ds(blk_row, BLKE // CH)], srcb)
      pltpu.sync_copy(dst_hbm.at[pl.ds(blk_row, BLKE // CH)], dstb)
      pltpu.async_copy(t_hbm.at[srcb.at[0]], rows0, sem_g0)
      pltpu.async_copy(adt_sh.at[dstb.at[0]], adch0, sem_a0)

      def pair_body(k2, carry2):
        c0 = 2 * k2
        pltpu.make_async_copy(t_hbm.at[srcb.at[0]], rows0, sem_g0).wait()
        pltpu.make_async_copy(adt_sh.at[dstb.at[0]], adch0, sem_a0).wait()
        pltpu.async_copy(t_hbm.at[srcb.at[c0 + 1]], rows1, sem_g1)
        pltpu.async_copy(adt_sh.at[dstb.at[c0 + 1]], adch1, sem_a1)

        @pl.when(k2 > 0)
        def _():
          pltpu.make_async_copy(out0, acc_sh.at[dstb.at[0]], sem_s0).wait()

        compute_chunk(rows0, adch0, out0)
        pltpu.async_copy(out0, acc_sh.at[dstb.at[c0]], sem_s0, add=True)

        @pl.when(k2 < BLKE // CH // 2 - 1)
        def _():
          pltpu.async_copy(t_hbm.at[srcb.at[c0 + 2]], rows0, sem_g0)
          pltpu.async_copy(adt_sh.at[dstb.at[c0 + 2]], adch0, sem_a0)

        pltpu.make_async_copy(t_hbm.at[srcb.at[0]], rows1, sem_g1).wait()
        pltpu.make_async_copy(adt_sh.at[dstb.at[0]], adch1, sem_a1).wait()

        @pl.when(k2 > 0)
        def _():
          pltpu.make_async_copy(out1, acc_sh.at[dstb.at[0]], sem_s1).wait()

        compute_chunk(rows1, adch1, out1)
        pltpu.async_copy(out1, acc_sh.at[dstb.at[c0 + 1]], sem_s1, add=True)
        return carry2

      lax.fori_loop(0, BLKE // CH // 2, pair_body, 0)
      pltpu.make_async_copy(out0, acc_sh.at[dstb.at[0]], sem_s0).wait()
      pltpu.make_async_copy(out1, acc_sh.at[dstb.at[0]], sem_s1).wait()
      return carry

    lax.fori_loop(0, NBLK, block_body, 0)
    plsc.subcore_barrier()
    pltpu.sync_copy(acc_sh.at[pl.ds(sid * ROWS_PT, ROWS_PT)],
                    acc_out.at[cid, pl.ds(sid * ROWS_PT, ROWS_PT)])

  return edge_kernel


def kernel(x, edge_index, params):
  src = edge_index[0].astype(jnp.int32)
  dst = edge_index[1].astype(jnp.int32)
  npad = E_PAD - E
  padv = N + (jnp.arange(npad, dtype=jnp.int32) % 2048)
  src2d = jnp.concatenate([src, padv]).reshape(E_PAD // CH, CH)
  dst2d = jnp.concatenate([dst, padv]).reshape(E_PAD // CH, CH)

  xp = jnp.zeros((NT, 10), jnp.float32).at[:N].set(x)
  edge_kernel = _make_edge_kernel()

  h = xp
  nl = len(params)
  for i, (W, a_src, a_dst, bias) in enumerate(params):
    dout = W.shape[1]
    w5 = jnp.zeros((W.shape[0], 5), jnp.float32).at[:, :dout].set(W)
    as5 = jnp.zeros((1, 5), jnp.float32).at[0, :dout].set(a_src)
    ad5 = jnp.zeros((1, 5), jnp.float32).at[0, :dout].set(a_dst)
    b5 = jnp.zeros((1, 5), jnp.float32).at[0, :dout].set(bias)
    t, adt = _prep_call(h, w5, as5, ad5)
    acc = edge_kernel(t, adt.reshape(NT), src2d, dst2d)
    h = _finalize_call(acc, t, b5, do_relu=(i < nl - 1))
  return h[:N, :1]


# speedup vs baseline: 232.4238x; 1.5561x over previous
import functools

import jax
import jax.numpy as jnp
from jax import lax
from jax.experimental import pallas as pl
from jax.experimental.pallas import tpu as pltpu
from jax.experimental.pallas import tpu_sc as plsc
from jax._src.pallas import core as pallas_core
from jax._src.pallas.mosaic import core as tpu_core

N = 100000
NT = 102400
E = 6400000
NC, NS, L = 2, 16, 16
NW = NC * NS
NBLK = 49
BLKE = 4096
CH = 128
EPT = NBLK * BLKE
E_PAD = NW * EPT
ROWS_PT = NT // NS
BN = 12800


def _prep_call(xt, wt5, as5, ad5):
  din = xt.shape[0]

  def body(x_ref, w_ref, as_ref, ad_ref, t_ref):
    h = jnp.dot(w_ref[...], x_ref[...], preferred_element_type=jnp.float32)
    asr = jnp.dot(as_ref[...], h, preferred_element_type=jnp.float32)
    adr = jnp.dot(ad_ref[...], h, preferred_element_type=jnp.float32)
    t_ref[...] = jnp.concatenate(
        [h, asr, adr, jnp.zeros((1, BN), jnp.float32)], axis=0)

  return pl.pallas_call(
      body,
      grid=(NT // BN,),
      in_specs=[
          pl.BlockSpec((din, BN), lambda i: (0, i)),
          pl.BlockSpec((5, din), lambda i: (0, 0)),
          pl.BlockSpec((1, 5), lambda i: (0, 0)),
          pl.BlockSpec((1, 5), lambda i: (0, 0)),
      ],
      out_specs=pl.BlockSpec((8, BN), lambda i: (0, i)),
      out_shape=jax.ShapeDtypeStruct((8, NT), jnp.float32),
  )(xt, wt5, as5, ad5)


def _finalize_call(acc, t8, b5, do_relu):
  def body(acc_ref, t_ref, b_ref, o_ref):
    a0 = acc_ref[0]
    a1 = acc_ref[1]
    h = t_ref[0:5, :]
    es = t_ref[5:6, :] + t_ref[6:7, :]
    ws = jnp.exp(jnp.maximum(es, es * 0.2))
    num = a0[0:5, :] + a1[0:5, :] + ws * h
    den = a0[5:6, :] + a1[5:6, :] + ws
    o = num / den + b_ref[...]
    if do_relu:
      o = jnp.maximum(o, 0.0)
    i = pl.program_id(0)
    cid = i * BN + lax.broadcasted_iota(jnp.int32, (5, BN), 1)
    o_ref[...] = jnp.where(cid < N, o, 0.0)

  return pl.pallas_call(
      body,
      grid=(NT // BN,),
      in_specs=[
          pl.BlockSpec((2, 6, BN), lambda i: (0, 0, i)),
          pl.BlockSpec((8, BN), lambda i: (0, i)),
          pl.BlockSpec((5, 1), lambda i: (0, 0)),
      ],
      out_specs=pl.BlockSpec((5, BN), lambda i: (0, i)),
      out_shape=jax.ShapeDtypeStruct((5, NT), jnp.float32),
  )(acc, t8, b5)


def _make_edge_kernel():
  mesh = plsc.VectorSubcoreMesh(core_axis_name="c", subcore_axis_name="s")
  shared = pallas_core.CoreMemorySpace(tpu_core.MemorySpace.VMEM_SHARED, mesh)

  @functools.partial(
      pl.kernel,
      out_type=jax.ShapeDtypeStruct((NC, 6, NT), jnp.float32),
      mesh=mesh,
      compiler_params=pltpu.CompilerParams(needs_layout_passes=False,
                                           use_tc_tiling_on_sc=False),
      scratch_types=(
          [
              pltpu.VMEM((BLKE // CH, CH), jnp.int32),
              pltpu.VMEM((BLKE // CH, CH), jnp.int32),
          ]
          + [pltpu.VMEM((8, CH), jnp.float32) for _ in range(4)]
          + [pltpu.VMEM((6, CH), jnp.float32) for _ in range(4)]
          + [pltpu.VMEM((ROWS_PT,), jnp.float32)]
          + [shared((NT,), jnp.float32) for _ in range(7)]
          + [shared((NT,), jnp.float32) for _ in range(6)]
          + [pltpu.SemaphoreType.DMA for _ in range(8)]
      ),
  )
  def edge_kernel(t8_hbm, src_hbm, dst_hbm, acc_out,
                  srcb, dstb, gb0, gb1, gb2, gb3, ob0, ob1, ob2, ob3, zbuf,
                  t0, t1, t2, t3, t4, t5, t6, a0, a1, a2, a3, a4, a5,
                  sg0, sg1, sg2, sg3, ss0, ss1, ss2, ss3):
    cid = lax.axis_index("c")
    sid = lax.axis_index("s")
    wid = cid * NS + sid
    iota = lax.iota(jnp.int32, L)
    zeros16 = jnp.zeros((L,), jnp.float32)
    tabs = [t0, t1, t2, t3, t4, t5, t6]
    accs = [a0, a1, a2, a3, a4, a5]
    gbs = [gb0, gb1, gb2, gb3]
    obs = [ob0, ob1, ob2, ob3]
    sgs = [sg0, sg1, sg2, sg3]
    sss = [ss0, ss1, ss2, ss3]

    for k in range(7):
      @pl.when(sid == k)
      def _(k=k):
        pltpu.sync_copy(t8_hbm.at[k], tabs[k])

    def zb_body(i, carry):
      plsc.store_scatter(zbuf, [i * L + iota], zeros16)
      return carry

    lax.fori_loop(0, ROWS_PT // L, zb_body, 0)
    for r in range(6):
      pltpu.sync_copy(zbuf, accs[r].at[pl.ds(sid * ROWS_PT, ROWS_PT)])
    plsc.subcore_barrier()

    row_base = wid * (EPT // CH)

    def fire_gathers(c, j):
      si = srcb.at[c]
      for k in range(6):
        pltpu.async_copy(tabs[k].at[si], gbs[j].at[k], sgs[j])
      pltpu.async_copy(tabs[6].at[dstb.at[c]], gbs[j].at[6], sgs[j])

    def wait_gathers(j):
      for k in range(7):
        pltpu.make_async_copy(tabs[0].at[srcb.at[0]], gbs[j].at[k],
                              sgs[j]).wait()

    def fire_scatters(c, j):
      di = dstb.at[c]
      for k in range(5):
        pltpu.async_copy(obs[j].at[k], accs[k].at[di], sss[j], add=True)
      pltpu.async_copy(obs[j].at[5], accs[5].at[di], sss[j], add=True)

    def wait_scatters(j):
      for k in range(6):
        pltpu.make_async_copy(obs[j].at[k], accs[0].at[dstb.at[0]],
                              sss[j]).wait()

    def compute_chunk(j):
      gb = gbs[j]
      ob = obs[j]
      for g in range(CH // L):
        sl = pl.ds(g * L, L)
        e = gb[5, sl] + gb[6, sl]
        w = jnp.exp(jnp.maximum(e, e * 0.2))
        ob[5, sl] = w
        for k in range(5):
          ob[k, sl] = gb[k, sl] * w

    def block_body(b, carry):
      blk_row = row_base + b * (BLKE // CH)
      pltpu.sync_copy(src_hbm.at[pl.ds(blk_row, BLKE // CH)], srcb)
      pltpu.sync_copy(dst_hbm.at[pl.ds(blk_row, BLKE // CH)], dstb)
      for j in range(4):
        fire_gathers(j, j)

      def quad_body(k4, carry2):
        for j in range(4):
          wait_gathers(j)

          @pl.when(k4 > 0)
          def _(j=j):
            wait_scatters(j)

          compute_chunk(j)
          fire_scatters(4 * k4 + j, j)

          @pl.when(k4 < BLKE // CH // 4 - 1)
          def _(k4=k4, j=j):
            fire_gathers(4 * (k4 + 1) + j, j)

        return carry2

      lax.fori_loop(0, BLKE // CH // 4, quad_body, 0)
      for j in range(4):
        wait_scatters(j)
      return carry

    lax.fori_loop(0, NBLK, block_body, 0)
    plsc.subcore_barrier()
    sl = pl.ds(sid * ROWS_PT, ROWS_PT)
    for r in range(6):
      pltpu.sync_copy(accs[r].at[sl], acc_out.at[cid, r, sl])

  return edge_kernel


def kernel(x, edge_index, params):
  src = edge_index[0].astype(jnp.int32)
  dst = edge_index[1].astype(jnp.int32)
  npad = E_PAD - E
  padv = N + (jnp.arange(npad, dtype=jnp.int32) % 2048)
  src2d = jnp.concatenate([src, padv]).reshape(E_PAD // CH, CH)
  dst2d = jnp.concatenate([dst, padv]).reshape(E_PAD // CH, CH)

  xt = jnp.zeros((10, NT), jnp.float32).at[:, :N].set(x.T)
  edge_kernel = _make_edge_kernel()

  nl = len(params)
  for i, (W, a_src, a_dst, bias) in enumerate(params):
    din, dout = W.shape
    wt5 = jnp.zeros((5, din), jnp.float32).at[:dout].set(W.T)
    as5 = jnp.zeros((1, 5), jnp.float32).at[0, :dout].set(a_src)
    ad5 = jnp.zeros((1, 5), jnp.float32).at[0, :dout].set(a_dst)
    b5 = jnp.zeros((5, 1), jnp.float32).at[:dout, 0].set(bias)
    t8 = _prep_call(xt, wt5, as5, ad5)
    acc = edge_kernel(t8, src2d, dst2d)
    xt = _finalize_call(acc, t8, b5, do_relu=(i < nl - 1))
  return xt[0, :N].reshape(N, 1)


# speedup vs baseline: 249.6364x; 1.0741x over previous
import functools

import jax
import jax.numpy as jnp
from jax import lax
from jax.experimental import pallas as pl
from jax.experimental.pallas import tpu as pltpu
from jax.experimental.pallas import tpu_sc as plsc
from jax._src.pallas import core as pallas_core
from jax._src.pallas.mosaic import core as tpu_core

N = 100000
NT = 102400
E = 6400000
NC, NS, L = 2, 16, 16
NW = NC * NS
NBLK = 49
BLKE = 4096
CH = 128
EPT = NBLK * BLKE
E_PAD = NW * EPT
ROWS_PT = NT // NS
BN = 12800


def _prep_call(xt, wt5, as5, ad5):
  din = xt.shape[0]

  def body(x_ref, w_ref, as_ref, ad_ref, t_ref):
    h = jnp.dot(w_ref[...], x_ref[...], preferred_element_type=jnp.float32)
    asr = jnp.dot(as_ref[...], h, preferred_element_type=jnp.float32)
    adr = jnp.dot(ad_ref[...], h, preferred_element_type=jnp.float32)
    t_ref[...] = jnp.concatenate(
        [h, asr, adr, jnp.zeros((1, BN), jnp.float32)], axis=0)

  return pl.pallas_call(
      body,
      grid=(NT // BN,),
      in_specs=[
          pl.BlockSpec((din, BN), lambda i: (0, i)),
          pl.BlockSpec((5, din), lambda i: (0, 0)),
          pl.BlockSpec((1, 5), lambda i: (0, 0)),
          pl.BlockSpec((1, 5), lambda i: (0, 0)),
      ],
      out_specs=pl.BlockSpec((8, BN), lambda i: (0, i)),
      out_shape=jax.ShapeDtypeStruct((8, NT), jnp.float32),
  )(xt, wt5, as5, ad5)


def _finalize_call(acc_t, t8, b5, do_relu):
  def body(acc_ref, t_ref, b_ref, o_ref):
    a0 = acc_ref[0]
    a1 = acc_ref[1]
    h = t_ref[0:5, :]
    es = t_ref[5:6, :] + t_ref[6:7, :]
    ws = jnp.exp(jnp.maximum(es, es * 0.2))
    num = a0[0:5, :] + a1[0:5, :] + ws * h
    den = a0[5:6, :] + a1[5:6, :] + ws
    o = num / den + b_ref[...]
    if do_relu:
      o = jnp.maximum(o, 0.0)
    i = pl.program_id(0)
    cid = i * BN + lax.broadcasted_iota(jnp.int32, (5, BN), 1)
    o_ref[...] = jnp.where(cid < N, o, 0.0)

  return pl.pallas_call(
      body,
      grid=(NT // BN,),
      in_specs=[
          pl.BlockSpec((2, 8, BN), lambda i: (0, 0, i)),
          pl.BlockSpec((8, BN), lambda i: (0, i)),
          pl.BlockSpec((5, 1), lambda i: (0, 0)),
      ],
      out_specs=pl.BlockSpec((5, BN), lambda i: (0, i)),
      out_shape=jax.ShapeDtypeStruct((5, NT), jnp.float32),
  )(acc_t, t8, b5)


def _make_edge_kernel():
  mesh = plsc.VectorSubcoreMesh(core_axis_name="c", subcore_axis_name="s")
  shared = pallas_core.CoreMemorySpace(tpu_core.MemorySpace.VMEM_SHARED, mesh)

  @functools.partial(
      pl.kernel,
      out_type=jax.ShapeDtypeStruct((NC, NT, 8), jnp.float32),
      mesh=mesh,
      compiler_params=pltpu.CompilerParams(needs_layout_passes=False,
                                           use_tc_tiling_on_sc=False),
      scratch_types=(
          [
              pltpu.VMEM((BLKE // CH, CH), jnp.int32),
              pltpu.VMEM((BLKE // CH, CH), jnp.int32),
          ]
          + [pltpu.VMEM((CH, 8), jnp.float32) for _ in range(4)]
          + [pltpu.VMEM((CH,), jnp.float32) for _ in range(4)]
          + [pltpu.VMEM((CH, 8), jnp.float32) for _ in range(4)]
          + [pltpu.VMEM((256, 8), jnp.float32)]
          + [
              shared((NT, 8), jnp.float32),
              shared((NT,), jnp.float32),
              shared((NT, 8), jnp.float32),
          ]
          + [pltpu.SemaphoreType.DMA for _ in range(8)]
      ),
  )
  def edge_kernel(trow_hbm, ad_hbm, src_hbm, dst_hbm, acc_out,
                  srcb, dstb, gb0, gb1, gb2, gb3, ac0, ac1, ac2, ac3,
                  ob0, ob1, ob2, ob3, zbuf,
                  trow_sh, ad_sh, acc_sh,
                  sg0, sg1, sg2, sg3, ss0, ss1, ss2, ss3):
    cid = lax.axis_index("c")
    sid = lax.axis_index("s")
    wid = cid * NS + sid
    iota = lax.iota(jnp.int32, L)
    zeros16 = jnp.zeros((L,), jnp.float32)
    col07 = lax.bitwise_and(iota, 7)
    shr3 = lax.shift_right_logical(iota, 3)
    gbs = [gb0, gb1, gb2, gb3]
    acs = [ac0, ac1, ac2, ac3]
    obs = [ob0, ob1, ob2, ob3]
    sgs = [sg0, sg1, sg2, sg3]
    sss = [ss0, ss1, ss2, ss3]

    tsl = pl.ds(sid * ROWS_PT, ROWS_PT)
    pltpu.sync_copy(trow_hbm.at[tsl], trow_sh.at[tsl])
    pltpu.sync_copy(ad_hbm.at[tsl], ad_sh.at[tsl])

    for p in range(16):
      plsc.store_scatter(zbuf, [shr3 + 2 * p, col07], zeros16)
      plsc.store_scatter(zbuf, [32 + shr3 + 2 * p, col07], zeros16)
      plsc.store_scatter(zbuf, [64 + shr3 + 2 * p, col07], zeros16)
      plsc.store_scatter(zbuf, [96 + shr3 + 2 * p, col07], zeros16)
      plsc.store_scatter(zbuf, [128 + shr3 + 2 * p, col07], zeros16)
      plsc.store_scatter(zbuf, [160 + shr3 + 2 * p, col07], zeros16)
      plsc.store_scatter(zbuf, [192 + shr3 + 2 * p, col07], zeros16)
      plsc.store_scatter(zbuf, [224 + shr3 + 2 * p, col07], zeros16)

    def zb_body(i, carry):
      pltpu.sync_copy(zbuf,
                      acc_sh.at[pl.ds(sid * ROWS_PT + i * 256, 256)])
      return carry

    lax.fori_loop(0, ROWS_PT // 256, zb_body, 0)
    plsc.subcore_barrier()

    row_base = wid * (EPT // CH)

    def fire_gathers(c, j):
      pltpu.async_copy(trow_sh.at[srcb.at[c]], gbs[j], sgs[j])
      pltpu.async_copy(ad_sh.at[dstb.at[c]], acs[j], sgs[j])

    def wait_gathers(j):
      pltpu.make_async_copy(trow_sh.at[srcb.at[0]], gbs[j], sgs[j]).wait()
      pltpu.make_async_copy(ad_sh.at[dstb.at[0]], acs[j], sgs[j]).wait()

    def fire_scatter(c, j):
      pltpu.async_copy(obs[j], acc_sh.at[dstb.at[c]], sss[j], add=True)

    def wait_scatter(j):
      pltpu.make_async_copy(obs[j], acc_sh.at[dstb.at[0]], sss[j]).wait()

    cols = [jnp.full((L,), k, jnp.int32) for k in range(6)]

    def compute_chunk(j):
      gb = gbs[j]
      ob = obs[j]
      ad = acs[j]
      for g in range(CH // L):
        rowv = g * L + iota
        asv = plsc.load_gather(gb, [rowv, cols[5]])
        adv = ad[pl.ds(g * L, L)]
        e = asv + adv
        w = jnp.exp(jnp.maximum(e, e * 0.2))
        plsc.store_scatter(ob, [rowv, cols[5]], w)
        for k in range(5):
          hv = plsc.load_gather(gb, [rowv, cols[k]])
          plsc.store_scatter(ob, [rowv, cols[k]], hv * w)

    def block_body(b, carry):
      blk_row = row_base + b * (BLKE // CH)
      pltpu.sync_copy(src_hbm.at[pl.ds(blk_row, BLKE // CH)], srcb)
      pltpu.sync_copy(dst_hbm.at[pl.ds(blk_row, BLKE // CH)], dstb)
      for j in range(4):
        fire_gathers(j, j)

      def quad_body(k4, carry2):
        for j in range(4):
          wait_gathers(j)

          @pl.when(k4 > 0)
          def _(j=j):
            wait_scatter(j)

          compute_chunk(j)
          fire_scatter(4 * k4 + j, j)

          @pl.when(k4 < BLKE // CH // 4 - 1)
          def _(k4=k4, j=j):
            fire_gathers(4 * (k4 + 1) + j, j)

        return carry2

      lax.fori_loop(0, BLKE // CH // 4, quad_body, 0)
      for j in range(4):
        wait_scatter(j)
      return carry

    lax.fori_loop(0, NBLK, block_body, 0)
    plsc.subcore_barrier()
    pltpu.sync_copy(acc_sh.at[tsl], acc_out.at[cid, tsl])

  return edge_kernel


def kernel(x, edge_index, params):
  src = edge_index[0].astype(jnp.int32)
  dst = edge_index[1].astype(jnp.int32)
  npad = E_PAD - E
  padv = N + (jnp.arange(npad, dtype=jnp.int32) % 2048)
  src2d = jnp.concatenate([src, padv]).reshape(E_PAD // CH, CH)
  dst2d = jnp.concatenate([dst, padv]).reshape(E_PAD // CH, CH)

  xt = jnp.zeros((10, NT), jnp.float32).at[:, :N].set(x.T)
  edge_kernel = _make_edge_kernel()

  nl = len(params)
  for i, (W, a_src, a_dst, bias) in enumerate(params):
    din, dout = W.shape
    wt5 = jnp.zeros((5, din), jnp.float32).at[:dout].set(W.T)
    as5 = jnp.zeros((1, 5), jnp.float32).at[0, :dout].set(a_src)
    ad5 = jnp.zeros((1, 5), jnp.float32).at[0, :dout].set(a_dst)
    b5 = jnp.zeros((5, 1), jnp.float32).at[:dout, 0].set(bias)
    t8 = _prep_call(xt, wt5, as5, ad5)
    trow = t8.T
    acc = edge_kernel(trow, t8[6], src2d, dst2d)
    acc_t = jnp.transpose(acc, (0, 2, 1))
    xt = _finalize_call(acc_t, t8, b5, do_relu=(i < nl - 1))
  return xt[0, :N].reshape(N, 1)


# speedup vs baseline: 285.9763x; 1.1456x over previous
import functools

import jax
import jax.numpy as jnp
from jax import lax
from jax.experimental import pallas as pl
from jax.experimental.pallas import tpu as pltpu
from jax.experimental.pallas import tpu_sc as plsc
from jax._src.pallas import core as pallas_core
from jax._src.pallas.mosaic import core as tpu_core

N = 100000
NT = 102400
E = 6400000
NC, NS, L = 2, 16, 16
NW = NC * NS
NBLK = 49
BLKE = 4096
CH = 128
EPT = NBLK * BLKE
E_PAD = NW * EPT
ROWS_PT = NT // NS
BN = 12800


def _prep_call(xt, wt5, as5, ad5):
  din = xt.shape[0]

  def body(x_ref, w_ref, as_ref, ad_ref, t_ref):
    h = jnp.dot(w_ref[...], x_ref[...], preferred_element_type=jnp.float32)
    asr = jnp.dot(as_ref[...], h, preferred_element_type=jnp.float32)
    adr = jnp.dot(ad_ref[...], h, preferred_element_type=jnp.float32)
    t_ref[...] = jnp.concatenate(
        [h, asr, adr, jnp.zeros((1, BN), jnp.float32)], axis=0)

  return pl.pallas_call(
      body,
      grid=(NT // BN,),
      in_specs=[
          pl.BlockSpec((din, BN), lambda i: (0, i)),
          pl.BlockSpec((5, din), lambda i: (0, 0)),
          pl.BlockSpec((1, 5), lambda i: (0, 0)),
          pl.BlockSpec((1, 5), lambda i: (0, 0)),
      ],
      out_specs=pl.BlockSpec((8, BN), lambda i: (0, i)),
      out_shape=jax.ShapeDtypeStruct((8, NT), jnp.float32),
  )(xt, wt5, as5, ad5)


def _finalize_call(acc_t, t8, b5, do_relu):
  def body(acc_ref, t_ref, b_ref, o_ref):
    a0 = acc_ref[0]
    a1 = acc_ref[1]
    h = t_ref[0:5, :]
    es = t_ref[5:6, :] + t_ref[6:7, :]
    ws = jnp.exp(jnp.maximum(es, es * 0.2))
    num = a0[0:5, :] + a1[0:5, :] + ws * h
    den = a0[5:6, :] + a1[5:6, :] + ws
    o = num / den + b_ref[...]
    if do_relu:
      o = jnp.maximum(o, 0.0)
    i = pl.program_id(0)
    cid = i * BN + lax.broadcasted_iota(jnp.int32, (5, BN), 1)
    o_ref[...] = jnp.where(cid < N, o, 0.0)

  return pl.pallas_call(
      body,
      grid=(NT // BN,),
      in_specs=[
          pl.BlockSpec((2, 8, BN), lambda i: (0, 0, i)),
          pl.BlockSpec((8, BN), lambda i: (0, i)),
          pl.BlockSpec((5, 1), lambda i: (0, 0)),
      ],
      out_specs=pl.BlockSpec((5, BN), lambda i: (0, i)),
      out_shape=jax.ShapeDtypeStruct((5, NT), jnp.float32),
  )(acc_t, t8, b5)


def _make_edge_kernel():
  mesh = plsc.VectorSubcoreMesh(core_axis_name="c", subcore_axis_name="s")
  shared = pallas_core.CoreMemorySpace(tpu_core.MemorySpace.VMEM_SHARED, mesh)

  @functools.partial(
      pl.kernel,
      out_type=jax.ShapeDtypeStruct((NC, NT, 8), jnp.float32),
      mesh=mesh,
      compiler_params=pltpu.CompilerParams(needs_layout_passes=False,
                                           use_tc_tiling_on_sc=False),
      scratch_types=(
          [
              pltpu.VMEM((BLKE // CH, CH), jnp.int32),
              pltpu.VMEM((BLKE // CH, CH), jnp.int32),
          ]
          + [pltpu.VMEM((CH, 8), jnp.float32) for _ in range(4)]
          + [pltpu.VMEM((CH,), jnp.float32) for _ in range(4)]
          + [pltpu.VMEM((CH, 8), jnp.float32) for _ in range(4)]
          + [pltpu.VMEM((256, 8), jnp.float32)]
          + [
              shared((NT, 8), jnp.float32),
              shared((NT,), jnp.float32),
              shared((NT, 8), jnp.float32),
          ]
          + [pltpu.SemaphoreType.DMA for _ in range(8)]
      ),
  )
  def edge_kernel(trow_hbm, ad_hbm, src_hbm, dst_hbm, acc_out,
                  srcb, dstb, gb0, gb1, gb2, gb3, ac0, ac1, ac2, ac3,
                  ob0, ob1, ob2, ob3, zbuf,
                  trow_sh, ad_sh, acc_sh,
                  sg0, sg1, sg2, sg3, ss0, ss1, ss2, ss3):
    cid = lax.axis_index("c")
    sid = lax.axis_index("s")
    wid = cid * NS + sid
    iota = lax.iota(jnp.int32, L)
    zeros16 = jnp.zeros((L,), jnp.float32)
    col07 = lax.bitwise_and(iota, 7)
    shr3 = lax.shift_right_logical(iota, 3)
    gbs = [gb0, gb1, gb2, gb3]
    acs = [ac0, ac1, ac2, ac3]
    obs = [ob0, ob1, ob2, ob3]
    sgs = [sg0, sg1, sg2, sg3]
    sss = [ss0, ss1, ss2, ss3]

    tsl = pl.ds(sid * ROWS_PT, ROWS_PT)
    pltpu.sync_copy(trow_hbm.at[tsl], trow_sh.at[tsl])
    pltpu.sync_copy(ad_hbm.at[tsl], ad_sh.at[tsl])

    for p in range(16):
      plsc.store_scatter(zbuf, [shr3 + 2 * p, col07], zeros16)
      plsc.store_scatter(zbuf, [32 + shr3 + 2 * p, col07], zeros16)
      plsc.store_scatter(zbuf, [64 + shr3 + 2 * p, col07], zeros16)
      plsc.store_scatter(zbuf, [96 + shr3 + 2 * p, col07], zeros16)
      plsc.store_scatter(zbuf, [128 + shr3 + 2 * p, col07], zeros16)
      plsc.store_scatter(zbuf, [160 + shr3 + 2 * p, col07], zeros16)
      plsc.store_scatter(zbuf, [192 + shr3 + 2 * p, col07], zeros16)
      plsc.store_scatter(zbuf, [224 + shr3 + 2 * p, col07], zeros16)

    def zb_body(i, carry):
      pltpu.sync_copy(zbuf,
                      acc_sh.at[pl.ds(sid * ROWS_PT + i * 256, 256)])
      return carry

    lax.fori_loop(0, ROWS_PT // 256, zb_body, 0)
    plsc.subcore_barrier()

    row_base = wid * (EPT // CH)

    def fire_gathers(c, j):
      pltpu.async_copy(trow_sh.at[srcb.at[c]], gbs[j], sgs[j])
      pltpu.async_copy(ad_sh.at[dstb.at[c]], acs[j], sgs[j])

    def wait_gathers(j):
      pltpu.make_async_copy(trow_sh.at[srcb.at[0]], gbs[j], sgs[j]).wait()
      pltpu.make_async_copy(ad_sh.at[dstb.at[0]], acs[j], sgs[j]).wait()

    def fire_scatter(c, j):
      pltpu.async_copy(obs[j], acc_sh.at[dstb.at[c]], sss[j], add=True)

    def wait_scatter(j):
      pltpu.make_async_copy(obs[j], acc_sh.at[dstb.at[0]], sss[j]).wait()

    cols = [jnp.full((L,), k, jnp.int32) for k in range(6)]

    def compute_chunk(j):
      gb = gbs[j]
      ob = obs[j]
      ad = acs[j]
      ws = []
      for g in range(CH // L):
        rowv = g * L + iota
        asv = plsc.load_gather(gb, [rowv, cols[5]])
        adv = ad[pl.ds(g * L, L)]
        e = asv + adv
        ws.append(jnp.exp(jnp.maximum(e, e * 0.2)))
      for g in range(CH // L):
        plsc.store_scatter(ob, [g * L + iota, cols[5]], ws[g])
      for k in range(5):
        for g in range(CH // L):
          rowv = g * L + iota
          hv = plsc.load_gather(gb, [rowv, cols[k]])
          plsc.store_scatter(ob, [rowv, cols[k]], hv * ws[g])

    def block_body(b, carry):
      blk_row = row_base + b * (BLKE // CH)
      pltpu.sync_copy(src_hbm.at[pl.ds(blk_row, BLKE // CH)], srcb)
      pltpu.sync_copy(dst_hbm.at[pl.ds(blk_row, BLKE // CH)], dstb)
      for j in range(4):
        fire_gathers(j, j)

      def quad_body(k4, carry2):
        for j in range(4):
          wait_gathers(j)

          @pl.when(k4 > 0)
          def _(j=j):
            wait_scatter(j)

          compute_chunk(j)
          fire_scatter(4 * k4 + j, j)

          @pl.when(k4 < BLKE // CH // 4 - 1)
          def _(k4=k4, j=j):
            fire_gathers(4 * (k4 + 1) + j, j)

        return carry2

      lax.fori_loop(0, BLKE // CH // 4, quad_body, 0)
      for j in range(4):
        wait_scatter(j)
      return carry

    lax.fori_loop(0, NBLK, block_body, 0)
    plsc.subcore_barrier()
    pltpu.sync_copy(acc_sh.at[tsl], acc_out.at[cid, tsl])

  return edge_kernel


def kernel(x, edge_index, params):
  src = edge_index[0].astype(jnp.int32)
  dst = edge_index[1].astype(jnp.int32)
  npad = E_PAD - E
  padv = N + (jnp.arange(npad, dtype=jnp.int32) % 2048)
  src2d = jnp.concatenate([src, padv]).reshape(E_PAD // CH, CH)
  dst2d = jnp.concatenate([dst, padv]).reshape(E_PAD // CH, CH)

  xt = jnp.zeros((10, NT), jnp.float32).at[:, :N].set(x.T)
  edge_kernel = _make_edge_kernel()

  nl = len(params)
  for i, (W, a_src, a_dst, bias) in enumerate(params):
    din, dout = W.shape
    wt5 = jnp.zeros((5, din), jnp.float32).at[:dout].set(W.T)
    as5 = jnp.zeros((1, 5), jnp.float32).at[0, :dout].set(a_src)
    ad5 = jnp.zeros((1, 5), jnp.float32).at[0, :dout].set(a_dst)
    b5 = jnp.zeros((5, 1), jnp.float32).at[:dout, 0].set(bias)
    t8 = _prep_call(xt, wt5, as5, ad5)
    trow = t8.T
    acc = edge_kernel(trow, t8[6], src2d, dst2d)
    acc_t = jnp.transpose(acc, (0, 2, 1))
    xt = _finalize_call(acc_t, t8, b5, do_relu=(i < nl - 1))
  return xt[0, :N].reshape(N, 1)
